# Initial kernel scaffold; baseline (speedup 1.0000x reference)
import jax
import jax.numpy as jnp
from jax.experimental import pallas as pl


def kernel(x, edge_index, W_self1, W_neigh1, b1, W_self2, W_neigh2, b2, W_reg, b_reg, W_fc1, b_fc1, W_fc2, b_fc2, W_flat, b_flat, W_gcn, b_gcn, prelu_a, W_bil, b_bil):
    raise NotImplementedError("write your pallas kernel here")


# speedup vs baseline: 3.1923x; 3.1923x over previous
import functools

import jax
import jax.numpy as jnp
from jax import lax
from jax.experimental import pallas as pl
from jax.experimental.pallas import tpu as pltpu
from jax.experimental.pallas import tpu_sc as plsc

N = 1000
E = 16000
IC = 128
NP = 5
NT = N * (1 + NP)
NHI = NT - N

NW = 32
EPAD = 16384
EPW = EPAD // NW
CPAD = 1000448
CSL = CPAD // 16
NPAD = 1024
NPW = NPAD // NW

_f32 = jnp.float32
_i32 = jnp.int32


def _sc_build_c_body(src_hbm, dst_hbm, val_hbm, zeros_hbm, out_hbm,
                     src_v, dst_v, val_v, fidx_v, zbuf_v, csh):
    cid = lax.axis_index("c")
    sid = lax.axis_index("s")
    wid = sid * 2 + cid
    pltpu.sync_copy(zeros_hbm.at[pl.ds(sid * CSL, CSL)], zbuf_v)
    pltpu.sync_copy(zbuf_v, csh.at[pl.ds(sid * CSL, CSL)])
    pltpu.sync_copy(src_hbm.at[pl.ds(wid * EPW, EPW)], src_v)
    pltpu.sync_copy(dst_hbm.at[pl.ds(wid * EPW, EPW)], dst_v)
    pltpu.sync_copy(val_hbm.at[pl.ds(wid * EPW, EPW)], val_v)
    for j in range(EPW // 16):
        sl = pl.ds(j * 16, 16)
        f = src_v[sl] * N + dst_v[sl]
        fidx_v[j // 8, pl.ds((j % 8) * 16, 16)] = f
    plsc.subcore_barrier()
    for ch in range(4):
        pltpu.sync_copy(val_v.at[pl.ds(ch * 128, 128)],
                        csh.at[fidx_v.at[ch]], add=True)
    plsc.subcore_barrier()
    pltpu.sync_copy(csh.at[pl.ds(sid * CSL, CSL)], zbuf_v)
    pltpu.sync_copy(zbuf_v, out_hbm.at[pl.ds(cid * CPAD + sid * CSL, CSL)])


def _sc_build_c(srcp, dstp, vals, zeros_c):
    mesh = plsc.VectorSubcoreMesh(core_axis_name="c", subcore_axis_name="s")
    kfn = pl.kernel(
        _sc_build_c_body,
        out_type=jax.ShapeDtypeStruct((2 * CPAD,), _f32),
        mesh=mesh,
        scratch_types=[
            pltpu.VMEM((EPW,), _i32),
            pltpu.VMEM((EPW,), _i32),
            pltpu.VMEM((EPW,), _f32),
            pltpu.VMEM((4, 128), _i32),
            pltpu.VMEM((CSL,), _f32),
            pltpu.VMEM_SHARED((CPAD,), _f32),
        ],
    )
    return kfn(srcp, dstp, vals, zeros_c)


def _dense_body(p_ref, x_ref, ws1_ref, wn1_ref, b1_ref, ws2_ref, wn2_ref,
                b2_ref, wreg_ref, breg_ref, wf1_ref, bf1_ref, wf2_ref,
                bf2_ref, wfl_ref, bfl_ref, noise_ref,
                c_ref, deg_ref, gen_ref, maskp_ref, dinv_ref):
    C = p_ref[0] + p_ref[1]
    c_ref[...] = C
    x = x_ref[...]
    ones = jnp.ones((N, 1), _f32)
    outdeg = jnp.dot(C, ones, preferred_element_type=_f32)
    indeg = lax.dot_general(C, ones, (((0,), (0,)), ((), ())),
                            preferred_element_type=_f32)
    deg_in = jnp.clip(indeg, 1.0, None)
    neigh1 = lax.dot_general(C, x, (((0,), (0,)), ((), ())),
                             preferred_element_type=_f32) / deg_in
    h1 = jnp.maximum(
        jnp.dot(x, ws1_ref[...], preferred_element_type=_f32)
        + jnp.dot(neigh1, wn1_ref[...], preferred_element_type=_f32)
        + b1_ref[...], 0.0)
    neigh2 = lax.dot_general(C, h1, (((0,), (0,)), ((), ())),
                             preferred_element_type=_f32) / deg_in
    z = (jnp.dot(h1, ws2_ref[...], preferred_element_type=_f32)
         + jnp.dot(neigh2, wn2_ref[...], preferred_element_type=_f32)
         + b2_ref[...])
    degree = jnp.maximum(
        jnp.dot(z, wreg_ref[...], preferred_element_type=_f32)
        + breg_ref[...], 0.0)
    deg_ref[...] = degree
    d = degree
    f = jnp.floor(d)
    frac = d - f
    odd = jnp.floor(f * 0.5) * 2.0 != f
    r = f + jnp.where((frac > 0.5) | ((frac == 0.5) & odd), 1.0, 0.0)
    deg_round = jnp.clip(r, 0.0, float(NP))
    k8 = lax.broadcasted_iota(_i32, (N, 8), 1).astype(_f32)
    maskp_ref[...] = (k8 < deg_round).astype(_f32)
    rowsum = outdeg + deg_round
    dinv_ref[...] = jnp.where(rowsum > 0, lax.rsqrt(rowsum), 0.0)
    g = z + noise_ref[...]
    g = jnp.maximum(jnp.dot(g, wf1_ref[...], preferred_element_type=_f32)
                    + bf1_ref[...], 0.0)
    g = jnp.maximum(jnp.dot(g, wf2_ref[...], preferred_element_type=_f32)
                    + bf2_ref[...], 0.0)
    gen_ref[...] = jnp.tanh(
        jnp.dot(g, wfl_ref[...], preferred_element_type=_f32) + bfl_ref[...])


def _dense(P2, x, Ws1, Wn1, b1, Ws2, Wn2, b2, Wreg, breg,
           Wf1, bf1, Wf2, bf2, Wfl, bfl, noise):
    out_shape = (
        jax.ShapeDtypeStruct((N, N), _f32),
        jax.ShapeDtypeStruct((N, 1), _f32),
        jax.ShapeDtypeStruct((N, NP * IC), _f32),
        jax.ShapeDtypeStruct((N, 8), _f32),
        jax.ShapeDtypeStruct((N, 1), _f32),
    )
    return pl.pallas_call(_dense_body, out_shape=out_shape)(
        P2, x, Ws1, Wn1, b1.reshape(1, -1), Ws2, Wn2, b2.reshape(1, -1),
        Wreg, breg.reshape(1, -1), Wf1, bf1.reshape(1, -1),
        Wf2, bf2.reshape(1, -1), Wfl, bfl.reshape(1, -1), noise)


def _yd_body(x_ref, gen5_ref, wg_ref, dinv_ref, yd_ref):
    wg = wg_ref[...]
    y0 = jnp.dot(x_ref[...], wg, preferred_element_type=_f32)
    y1 = jnp.dot(gen5_ref[...], wg, preferred_element_type=_f32)
    yc = jnp.concatenate([y0, y1], axis=0)
    dcol = jnp.concatenate(
        [dinv_ref[...], jnp.zeros((NT - N, 1), _f32)], axis=0)
    yd_ref[...] = jnp.concatenate(
        [yc, dcol, jnp.zeros((NT, 63), _f32)], axis=1)


def _yd(x, gen5, Wg, dinv):
    return pl.pallas_call(
        _yd_body, out_shape=jax.ShapeDtypeStruct((NT, 128), _f32))(
            x, gen5, Wg, dinv)


def _walk_body(cur_ref, c_ref, m_ref, g_ref, o_ref):
    i = pl.program_id(0)
    rows, mrows, ivals = [], [], []
    for r in range(8):
        cv = cur_ref[i * 8 + r]
        cc = jnp.minimum(cv, N - 1)
        rows.append(c_ref[pl.ds(cc, 1), :])
        mrows.append(m_ref[pl.ds(cc, 1), :])
        ivals.append(cv)
    crows = jnp.concatenate(rows, axis=0)
    mr = jnp.concatenate(mrows, axis=0)
    iv = jnp.concatenate([v.reshape(1, 1) for v in ivals], axis=0)
    valid = iv < N
    lane_lo = lax.broadcasted_iota(_i32, (8, N), 1)
    onehot = (lane_lo == iv).astype(_f32)
    low_w = jnp.where(valid, crows + onehot, 0.0)
    low = jnp.log(low_w + 1e-12) + g_ref[:, :N]
    ghi = g_ref[:, N:]
    lane_hi = lax.broadcasted_iota(_i32, (8, NHI), 1)
    base = jnp.log(jnp.zeros((8, 1), _f32) + 1e-12)
    val_hi = base + ghi
    for k in range(NP):
        cond = valid & (lane_hi == NP * iv + k)
        mk = jnp.log(mr[:, k:k + 1] + 1e-12)
        val_hi = jnp.where(cond, mk + ghi, val_hi)
    self_log = jnp.log(jnp.ones((8, 1), _f32) + 1e-12)
    cond2 = jnp.logical_not(valid) & (lane_hi == iv - N)
    val_hi = jnp.where(cond2, self_log + ghi, val_hi)
    ml = jnp.max(low, axis=1, keepdims=True)
    mh = jnp.max(val_hi, axis=1, keepdims=True)
    al = jnp.min(jnp.where(low == ml, lane_lo, NT), axis=1, keepdims=True)
    ah = jnp.min(jnp.where(val_hi == mh, lane_hi, NT), axis=1, keepdims=True)
    o_ref[0, :, :] = jnp.where(ml >= mh, al, ah + N)


def _walk_step(cur, C, maskp, G):
    grid_spec = pltpu.PrefetchScalarGridSpec(
        num_scalar_prefetch=1,
        grid=(N // 8,),
        in_specs=[
            pl.BlockSpec((N, N), lambda i, *_: (0, 0)),
            pl.BlockSpec((N, 8), lambda i, *_: (0, 0)),
            pl.BlockSpec((8, NT), lambda i, *_: (i, 0)),
        ],
        out_specs=pl.BlockSpec((1, 8, 1), lambda i, *_: (i, 0, 0)),
    )
    out = pl.pallas_call(
        _walk_body, grid_spec=grid_spec,
        out_shape=jax.ShapeDtypeStruct((N // 8, 8, 1), _i32))(
            cur, C, maskp, G)
    return out.reshape(N)


def _sc_gather_body(curs_hbm, yd_hbm, cflat_hbm, ydsub_hbm, cvals_hbm,
                    idx3_v, rows_v, fidx_v, cv_v, sem):
    cid = lax.axis_index("c")
    sid = lax.axis_index("s")
    wid = sid * 2 + cid
    bs = wid * NPW
    for v in range(3):
        pltpu.sync_copy(curs_hbm.at[pl.ds(v * NPAD + bs, NPW)], idx3_v.at[v])
    for v in range(3):
        pltpu.async_copy(yd_hbm.at[idx3_v.at[v]], rows_v, sem).wait()
        pltpu.sync_copy(rows_v, ydsub_hbm.at[v, pl.ds(bs, NPW)])
    iota = lax.broadcasted_iota(_i32, (16,), 0)
    for h in range(2):
        s0 = iota + (bs + 16 * h)
        svals = [s0] + [idx3_v[v, pl.ds(16 * h, 16)] for v in range(3)]
        cl = [jnp.minimum(s, N - 1) for s in svals]
        p = 0
        for v in range(3):
            for u in range(4):
                fidx_v[p, pl.ds(16 * h, 16)] = cl[v] * N + cl[u]
                p += 1
    handles = [pltpu.async_copy(cflat_hbm.at[fidx_v.at[p]], cv_v.at[p], sem)
               for p in range(12)]
    for hd in handles:
        hd.wait()
    for p in range(12):
        pltpu.sync_copy(cv_v.at[p], cvals_hbm.at[pl.ds(p * NPAD + bs, NPW)])


def _sc_gather(curs, yd, cflat):
    mesh = plsc.VectorSubcoreMesh(core_axis_name="c", subcore_axis_name="s")
    kfn = pl.kernel(
        _sc_gather_body,
        out_type=(jax.ShapeDtypeStruct((3, NPAD, 128), _f32),
                  jax.ShapeDtypeStruct((12 * NPAD,), _f32)),
        mesh=mesh,
        scratch_types=[
            pltpu.VMEM((3, NPW), _i32),
            pltpu.VMEM((NPW, 128), _f32),
            pltpu.VMEM((12, NPW), _i32),
            pltpu.VMEM((12, NPW), _f32),
            pltpu.SemaphoreType.DMA,
        ],
    )
    return kfn(curs, yd, cflat)


def _final_body(y0_ref, dg0_ref, ys_ref, dgs_ref, cv_ref, s_ref,
                wb_ref, bb_ref, bg_ref, pa_ref, out_ref):
    bg = bg_ref[...]
    pa = pa_ref[0, 0]
    s0 = lax.broadcasted_iota(_i32, (N, 1), 0)
    svals = [s0] + [s_ref[v].astype(_i32) for v in range(3)]
    dgs = [dg0_ref[...]] + [dgs_ref[v] for v in range(3)]
    ys = [y0_ref[...]] + [ys_ref[v] for v in range(3)]
    hsum = jnp.zeros((N, 64), _f32)
    for u in range(4):
        hg = jnp.broadcast_to(bg, (N, 64))
        for v in range(3):
            cval = cv_ref[:, v * 4 + u:v * 4 + u + 1]
            eq = (svals[u] == svals[v]).astype(_f32)
            A = dgs[u] * cval * dgs[v] + eq
            hg = hg + A * ys[v]
        hsum = hsum + jnp.where(hg > 0, hg, pa * hg)
    c = hsum * 0.25
    hmv_pre = ys[3] + bg
    h_mv = jnp.where(hmv_pre > 0, hmv_pre, pa * hmv_pre)
    m = jnp.dot(h_mv, wb_ref[...], preferred_element_type=_f32)
    out_ref[...] = (jnp.sum(m * c, axis=1, keepdims=True) + bb_ref[0, 0])


def _final(y0, dg0, ys3, dgs3, cvals, s3, Wb0, bb, bg, pa):
    return pl.pallas_call(
        _final_body, out_shape=jax.ShapeDtypeStruct((N, 1), _f32))(
            y0, dg0, ys3, dgs3, cvals, s3,
            Wb0, bb.reshape(1, 1), bg.reshape(1, -1), pa.reshape(1, 1))


def kernel(x, edge_index, W_self1, W_neigh1, b1, W_self2, W_neigh2, b2,
           W_reg, b_reg, W_fc1, b_fc1, W_fc2, b_fc2, W_flat, b_flat,
           W_gcn, b_gcn, prelu_a, W_bil, b_bil):
    src = edge_index[0].astype(_i32)
    dst = edge_index[1].astype(_i32)
    pad = EPAD - E
    srcp = jnp.concatenate([src, jnp.zeros((pad,), _i32)])
    dstp = jnp.concatenate([dst, jnp.zeros((pad,), _i32)])
    vals = jnp.concatenate([jnp.ones((E,), _f32), jnp.zeros((pad,), _f32)])
    zeros_c = jnp.zeros((CPAD,), _f32)

    partials = _sc_build_c(srcp, dstp, vals, zeros_c)
    P2 = partials.reshape(2, CPAD)[:, :N * N].reshape(2, N, N)

    noise = jax.random.normal(jax.random.key(7), (N, 64), _f32)
    C, degree, gen_feat, maskp, d_inv = _dense(
        P2, x, W_self1, W_neigh1, b1, W_self2, W_neigh2, b2, W_reg, b_reg,
        W_fc1, b_fc1, W_fc2, b_fc2, W_flat, b_flat, noise)

    gen5 = gen_feat.reshape(NP * N, IC)
    yd = _yd(x, gen5, W_gcn, d_inv)

    cur = jnp.arange(N, dtype=_i32)
    curs = []
    for t in range(1, 4):
        G = jax.random.gumbel(
            jax.random.fold_in(jax.random.key(42), t), (N, NT), _f32)
        cur = _walk_step(cur, C, maskp, G)
        curs.append(cur)

    curs_pad = jnp.concatenate(
        [jnp.concatenate([c, jnp.zeros((NPAD - N,), _i32)]) for c in curs])
    cflat = C.reshape(N * N)
    ydsub, cvals_flat = _sc_gather(curs_pad, yd, cflat)
    cvals = cvals_flat.reshape(12, NPAD)

    y0 = yd[:N, :64]
    dg0 = d_inv
    ys3 = ydsub[:, :N, :64]
    dgs3 = ydsub[:, :N, 64:65]
    s3 = jnp.stack(curs).reshape(3, N, 1)[:, :, :]
    logits = _final(y0, dg0, ys3, dgs3, cvals[:, :N].T, s3,
                    W_bil[0], b_bil, b_gcn, prelu_a)
    return degree, gen_feat, logits


# speedup vs baseline: 3.5622x; 1.1159x over previous
import functools

import jax
import jax.numpy as jnp
from jax import lax
from jax.experimental import pallas as pl
from jax.experimental.pallas import tpu as pltpu
from jax.experimental.pallas import tpu_sc as plsc

N = 1000
E = 16000
IC = 128
NP = 5
NT = N * (1 + NP)
NHI = NT - N

NW = 32
EPAD = 16384
EPW = EPAD // NW
CPAD = 1000448
CSL = CPAD // 16
NPAD = 1024
NPW = NPAD // NW

_f32 = jnp.float32
_i32 = jnp.int32


def _sc_build_c_body(src_hbm, dst_hbm, val_hbm, zeros_hbm, out_hbm,
                     src_v, dst_v, val_v, fidx_v, zbuf_v, csh):
    cid = lax.axis_index("c")
    sid = lax.axis_index("s")
    wid = sid * 2 + cid
    pltpu.sync_copy(zeros_hbm.at[pl.ds(sid * CSL, CSL)], zbuf_v)
    pltpu.sync_copy(zbuf_v, csh.at[pl.ds(sid * CSL, CSL)])
    pltpu.sync_copy(src_hbm.at[pl.ds(wid * EPW, EPW)], src_v)
    pltpu.sync_copy(dst_hbm.at[pl.ds(wid * EPW, EPW)], dst_v)
    pltpu.sync_copy(val_hbm.at[pl.ds(wid * EPW, EPW)], val_v)
    for j in range(EPW // 16):
        sl = pl.ds(j * 16, 16)
        f = src_v[sl] * N + dst_v[sl]
        fidx_v[j // 8, pl.ds((j % 8) * 16, 16)] = f
    plsc.subcore_barrier()
    for ch in range(4):
        pltpu.sync_copy(val_v.at[pl.ds(ch * 128, 128)],
                        csh.at[fidx_v.at[ch]], add=True)
    plsc.subcore_barrier()
    pltpu.sync_copy(csh.at[pl.ds(sid * CSL, CSL)], zbuf_v)
    pltpu.sync_copy(zbuf_v, out_hbm.at[pl.ds(cid * CPAD + sid * CSL, CSL)])


def _sc_build_c(srcp, dstp, vals, zeros_c):
    mesh = plsc.VectorSubcoreMesh(core_axis_name="c", subcore_axis_name="s")
    kfn = pl.kernel(
        _sc_build_c_body,
        out_type=jax.ShapeDtypeStruct((2 * CPAD,), _f32),
        mesh=mesh,
        scratch_types=[
            pltpu.VMEM((EPW,), _i32),
            pltpu.VMEM((EPW,), _i32),
            pltpu.VMEM((EPW,), _f32),
            pltpu.VMEM((4, 128), _i32),
            pltpu.VMEM((CSL,), _f32),
            pltpu.VMEM_SHARED((CPAD,), _f32),
        ],
    )
    return kfn(srcp, dstp, vals, zeros_c)


def _dense_body(p_ref, x_ref, ws1_ref, wn1_ref, b1_ref, ws2_ref, wn2_ref,
                b2_ref, wreg_ref, breg_ref, wf1_ref, bf1_ref, wf2_ref,
                bf2_ref, wfl_ref, bfl_ref, noise_ref,
                c_ref, deg_ref, gen_ref, maskp_ref, dinv_ref):
    C = p_ref[0] + p_ref[1]
    c_ref[...] = C
    x = x_ref[...]
    ones = jnp.ones((N, 1), _f32)
    outdeg = jnp.dot(C, ones, preferred_element_type=_f32)
    indeg = lax.dot_general(C, ones, (((0,), (0,)), ((), ())),
                            preferred_element_type=_f32)
    deg_in = jnp.clip(indeg, 1.0, None)
    neigh1 = lax.dot_general(C, x, (((0,), (0,)), ((), ())),
                             preferred_element_type=_f32) / deg_in
    h1 = jnp.maximum(
        jnp.dot(x, ws1_ref[...], preferred_element_type=_f32)
        + jnp.dot(neigh1, wn1_ref[...], preferred_element_type=_f32)
        + b1_ref[...], 0.0)
    neigh2 = lax.dot_general(C, h1, (((0,), (0,)), ((), ())),
                             preferred_element_type=_f32) / deg_in
    z = (jnp.dot(h1, ws2_ref[...], preferred_element_type=_f32)
         + jnp.dot(neigh2, wn2_ref[...], preferred_element_type=_f32)
         + b2_ref[...])
    degree = jnp.maximum(
        jnp.dot(z, wreg_ref[...], preferred_element_type=_f32)
        + breg_ref[...], 0.0)
    deg_ref[...] = degree
    d = degree
    f = jnp.floor(d)
    frac = d - f
    odd = jnp.floor(f * 0.5) * 2.0 != f
    r = f + jnp.where((frac > 0.5) | ((frac == 0.5) & odd), 1.0, 0.0)
    deg_round = jnp.clip(r, 0.0, float(NP))
    k8 = lax.broadcasted_iota(_i32, (N, 8), 1).astype(_f32)
    maskp_ref[...] = (k8 < deg_round).astype(_f32)
    rowsum = outdeg + deg_round
    dinv_ref[...] = jnp.where(rowsum > 0, lax.rsqrt(rowsum), 0.0)
    g = z + noise_ref[...]
    g = jnp.maximum(jnp.dot(g, wf1_ref[...], preferred_element_type=_f32)
                    + bf1_ref[...], 0.0)
    g = jnp.maximum(jnp.dot(g, wf2_ref[...], preferred_element_type=_f32)
                    + bf2_ref[...], 0.0)
    gen_ref[...] = jnp.tanh(
        jnp.dot(g, wfl_ref[...], preferred_element_type=_f32) + bfl_ref[...])


def _dense(P2, x, Ws1, Wn1, b1, Ws2, Wn2, b2, Wreg, breg,
           Wf1, bf1, Wf2, bf2, Wfl, bfl, noise):
    out_shape = (
        jax.ShapeDtypeStruct((N, N), _f32),
        jax.ShapeDtypeStruct((N, 1), _f32),
        jax.ShapeDtypeStruct((N, NP * IC), _f32),
        jax.ShapeDtypeStruct((N, 8), _f32),
        jax.ShapeDtypeStruct((N, 1), _f32),
    )
    return pl.pallas_call(_dense_body, out_shape=out_shape)(
        P2, x, Ws1, Wn1, b1.reshape(1, -1), Ws2, Wn2, b2.reshape(1, -1),
        Wreg, breg.reshape(1, -1), Wf1, bf1.reshape(1, -1),
        Wf2, bf2.reshape(1, -1), Wfl, bfl.reshape(1, -1), noise)


def _yd_body(x_ref, gen5_ref, wg_ref, dinv_ref, yd_ref):
    wg = wg_ref[...]
    y0 = jnp.dot(x_ref[...], wg, preferred_element_type=_f32)
    y1 = jnp.dot(gen5_ref[...], wg, preferred_element_type=_f32)
    yc = jnp.concatenate([y0, y1], axis=0)
    dcol = jnp.concatenate(
        [dinv_ref[...], jnp.zeros((NT - N, 1), _f32)], axis=0)
    yd_ref[...] = jnp.concatenate(
        [yc, dcol, jnp.zeros((NT, 63), _f32)], axis=1)


def _yd(x, gen5, Wg, dinv):
    return pl.pallas_call(
        _yd_body, out_shape=jax.ShapeDtypeStruct((NT, 128), _f32))(
            x, gen5, Wg, dinv)


def _walk_body(cur_ref, c_ref, m_ref, g_ref, o_ref):
    i = pl.program_id(0)
    rows, mrows, ivals = [], [], []
    for r in range(8):
        cv = cur_ref[i * 8 + r]
        cc = jnp.minimum(cv, N - 1)
        rows.append(c_ref[pl.ds(cc, 1), :])
        mrows.append(m_ref[pl.ds(cc, 1), :])
        ivals.append(cv)
    crows = jnp.concatenate(rows, axis=0)
    mr = jnp.concatenate(mrows, axis=0)
    iv = jnp.concatenate([v.reshape(1, 1) for v in ivals], axis=0)
    valid = iv < N
    lane_lo = lax.broadcasted_iota(_i32, (8, N), 1)
    onehot = (lane_lo == iv).astype(_f32)
    low_w = jnp.where(valid, crows + onehot, 0.0)
    low = jnp.log(low_w + 1e-12) + g_ref[:, :N]
    ghi = g_ref[:, N:]
    lane_hi = lax.broadcasted_iota(_i32, (8, NHI), 1)
    dr = jnp.sum(mr, axis=1, keepdims=True).astype(_i32)
    log0 = jnp.log(jnp.zeros((8, 1), _f32) + 1e-12)
    log1 = jnp.log(jnp.ones((8, 1), _f32) + 1e-12)
    rel = lane_hi - NP * iv
    in_mend = valid & (rel >= 0) & (rel < dr)
    in_self = jnp.logical_not(valid) & (lane_hi == iv - N)
    val_hi = ghi + jnp.where(in_mend | in_self, log1, log0)
    ml = jnp.max(low, axis=1, keepdims=True)
    mh = jnp.max(val_hi, axis=1, keepdims=True)
    al = jnp.min(jnp.where(low == ml, lane_lo, NT), axis=1, keepdims=True)
    ah = jnp.min(jnp.where(val_hi == mh, lane_hi, NT), axis=1, keepdims=True)
    o_ref[0, :, :] = jnp.where(ml >= mh, al, ah + N)


def _walk_step(cur, C, maskp, G):
    grid_spec = pltpu.PrefetchScalarGridSpec(
        num_scalar_prefetch=1,
        grid=(N // 8,),
        in_specs=[
            pl.BlockSpec((N, N), lambda i, *_: (0, 0)),
            pl.BlockSpec((N, 8), lambda i, *_: (0, 0)),
            pl.BlockSpec((8, NT), lambda i, *_: (i, 0)),
        ],
        out_specs=pl.BlockSpec((1, 8, 1), lambda i, *_: (i, 0, 0)),
    )
    out = pl.pallas_call(
        _walk_body, grid_spec=grid_spec,
        out_shape=jax.ShapeDtypeStruct((N // 8, 8, 1), _i32))(
            cur, C, maskp, G)
    return out.reshape(N)


def _sc_gather_body(curs_hbm, yd_hbm, cflat_hbm, ydsub_hbm, cvals_hbm,
                    idx3_v, rows_v, fidx_v, cv_v, sem):
    cid = lax.axis_index("c")
    sid = lax.axis_index("s")
    wid = sid * 2 + cid
    bs = wid * NPW
    for v in range(3):
        pltpu.sync_copy(curs_hbm.at[pl.ds(v * NPAD + bs, NPW)], idx3_v.at[v])
    for v in range(3):
        pltpu.async_copy(yd_hbm.at[idx3_v.at[v]], rows_v, sem).wait()
        pltpu.sync_copy(rows_v, ydsub_hbm.at[v, pl.ds(bs, NPW)])
    iota = lax.broadcasted_iota(_i32, (16,), 0)
    for h in range(2):
        s0 = iota + (bs + 16 * h)
        svals = [s0] + [idx3_v[v, pl.ds(16 * h, 16)] for v in range(3)]
        cl = [jnp.minimum(s, N - 1) for s in svals]
        p = 0
        for v in range(3):
            for u in range(4):
                fidx_v[p, pl.ds(16 * h, 16)] = cl[v] * N + cl[u]
                p += 1
    handles = [pltpu.async_copy(cflat_hbm.at[fidx_v.at[p]], cv_v.at[p], sem)
               for p in range(12)]
    for hd in handles:
        hd.wait()
    for p in range(12):
        pltpu.sync_copy(cv_v.at[p], cvals_hbm.at[pl.ds(p * NPAD + bs, NPW)])


def _sc_gather(curs, yd, cflat):
    mesh = plsc.VectorSubcoreMesh(core_axis_name="c", subcore_axis_name="s")
    kfn = pl.kernel(
        _sc_gather_body,
        out_type=(jax.ShapeDtypeStruct((3, NPAD, 128), _f32),
                  jax.ShapeDtypeStruct((12 * NPAD,), _f32)),
        mesh=mesh,
        scratch_types=[
            pltpu.VMEM((3, NPW), _i32),
            pltpu.VMEM((NPW, 128), _f32),
            pltpu.VMEM((12, NPW), _i32),
            pltpu.VMEM((12, NPW), _f32),
            pltpu.SemaphoreType.DMA,
        ],
    )
    return kfn(curs, yd, cflat)


def _final_body(y0_ref, dg0_ref, ys_ref, dgs_ref, cv_ref, s_ref,
                wb_ref, bb_ref, bg_ref, pa_ref, out_ref):
    bg = bg_ref[...]
    pa = pa_ref[0, 0]
    s0 = lax.broadcasted_iota(_i32, (N, 1), 0)
    svals = [s0] + [s_ref[v].astype(_i32) for v in range(3)]
    dgs = [dg0_ref[...]] + [dgs_ref[v] for v in range(3)]
    ys = [y0_ref[...]] + [ys_ref[v] for v in range(3)]
    hsum = jnp.zeros((N, 64), _f32)
    for u in range(4):
        hg = jnp.broadcast_to(bg, (N, 64))
        for v in range(3):
            cval = cv_ref[:, v * 4 + u:v * 4 + u + 1]
            eq = (svals[u] == svals[v]).astype(_f32)
            A = dgs[u] * cval * dgs[v] + eq
            hg = hg + A * ys[v]
        hsum = hsum + jnp.where(hg > 0, hg, pa * hg)
    c = hsum * 0.25
    hmv_pre = ys[3] + bg
    h_mv = jnp.where(hmv_pre > 0, hmv_pre, pa * hmv_pre)
    m = jnp.dot(h_mv, wb_ref[...], preferred_element_type=_f32)
    out_ref[...] = (jnp.sum(m * c, axis=1, keepdims=True) + bb_ref[0, 0])


def _final(y0, dg0, ys3, dgs3, cvals, s3, Wb0, bb, bg, pa):
    return pl.pallas_call(
        _final_body, out_shape=jax.ShapeDtypeStruct((N, 1), _f32))(
            y0, dg0, ys3, dgs3, cvals, s3,
            Wb0, bb.reshape(1, 1), bg.reshape(1, -1), pa.reshape(1, 1))


def kernel(x, edge_index, W_self1, W_neigh1, b1, W_self2, W_neigh2, b2,
           W_reg, b_reg, W_fc1, b_fc1, W_fc2, b_fc2, W_flat, b_flat,
           W_gcn, b_gcn, prelu_a, W_bil, b_bil):
    src = edge_index[0].astype(_i32)
    dst = edge_index[1].astype(_i32)
    pad = EPAD - E
    srcp = jnp.concatenate([src, jnp.zeros((pad,), _i32)])
    dstp = jnp.concatenate([dst, jnp.zeros((pad,), _i32)])
    vals = jnp.concatenate([jnp.ones((E,), _f32), jnp.zeros((pad,), _f32)])
    zeros_c = jnp.zeros((CPAD,), _f32)

    partials = _sc_build_c(srcp, dstp, vals, zeros_c)
    P2 = partials.reshape(2, CPAD)[:, :N * N].reshape(2, N, N)

    noise = jax.random.normal(jax.random.key(7), (N, 64), _f32)
    C, degree, gen_feat, maskp, d_inv = _dense(
        P2, x, W_self1, W_neigh1, b1, W_self2, W_neigh2, b2, W_reg, b_reg,
        W_fc1, b_fc1, W_fc2, b_fc2, W_flat, b_flat, noise)

    gen5 = gen_feat.reshape(NP * N, IC)
    yd = _yd(x, gen5, W_gcn, d_inv)

    cur = jnp.arange(N, dtype=_i32)
    curs = []
    for t in range(1, 4):
        G = jax.random.gumbel(
            jax.random.fold_in(jax.random.key(42), t), (N, NT), _f32)
        cur = _walk_step(cur, C, maskp, G)
        curs.append(cur)

    curs_pad = jnp.concatenate(
        [jnp.concatenate([c, jnp.zeros((NPAD - N,), _i32)]) for c in curs])
    cflat = C.reshape(N * N)
    ydsub, cvals_flat = _sc_gather(curs_pad, yd, cflat)
    cvals = cvals_flat.reshape(12, NPAD)

    y0 = yd[:N, :64]
    dg0 = d_inv
    ys3 = ydsub[:, :N, :64]
    dgs3 = ydsub[:, :N, 64:65]
    s3 = jnp.stack(curs).reshape(3, N, 1)[:, :, :]
    logits = _final(y0, dg0, ys3, dgs3, cvals[:, :N].T, s3,
                    W_bil[0], b_bil, b_gcn, prelu_a)
    return degree, gen_feat, logits


# speedup vs baseline: 4.8072x; 1.3495x over previous
import functools

import jax
import jax.numpy as jnp
from jax import lax
from jax.experimental import pallas as pl
from jax.experimental.pallas import tpu as pltpu
from jax.experimental.pallas import tpu_sc as plsc

N = 1000
E = 16000
IC = 128
NP = 5
NT = N * (1 + NP)
NHI = NT - N

NW = 32
EPAD = 16384
EPW = EPAD // NW
CPAD = 1000448
CSL = CPAD // 16
NPAD = 1024
NPW = NPAD // NW

_f32 = jnp.float32
_i32 = jnp.int32


def _sc_build_c_body(src_hbm, dst_hbm, val_hbm, zeros_hbm, out_hbm,
                     src_v, dst_v, val_v, fidx_v, zbuf_v, csh):
    cid = lax.axis_index("c")
    sid = lax.axis_index("s")
    wid = sid * 2 + cid
    pltpu.sync_copy(zeros_hbm.at[pl.ds(sid * CSL, CSL)], zbuf_v)
    pltpu.sync_copy(zbuf_v, csh.at[pl.ds(sid * CSL, CSL)])
    pltpu.sync_copy(src_hbm.at[pl.ds(wid * EPW, EPW)], src_v)
    pltpu.sync_copy(dst_hbm.at[pl.ds(wid * EPW, EPW)], dst_v)
    pltpu.sync_copy(val_hbm.at[pl.ds(wid * EPW, EPW)], val_v)
    for j in range(EPW // 16):
        sl = pl.ds(j * 16, 16)
        f = src_v[sl] * N + dst_v[sl]
        fidx_v[j // 8, pl.ds((j % 8) * 16, 16)] = f
    plsc.subcore_barrier()
    for ch in range(4):
        pltpu.sync_copy(val_v.at[pl.ds(ch * 128, 128)],
                        csh.at[fidx_v.at[ch]], add=True)
    plsc.subcore_barrier()
    pltpu.sync_copy(csh.at[pl.ds(sid * CSL, CSL)], zbuf_v)
    pltpu.sync_copy(zbuf_v, out_hbm.at[pl.ds(cid * CPAD + sid * CSL, CSL)])


def _sc_build_c(srcp, dstp, vals, zeros_c):
    mesh = plsc.VectorSubcoreMesh(core_axis_name="c", subcore_axis_name="s")
    kfn = pl.kernel(
        _sc_build_c_body,
        out_type=jax.ShapeDtypeStruct((2 * CPAD,), _f32),
        mesh=mesh,
        scratch_types=[
            pltpu.VMEM((EPW,), _i32),
            pltpu.VMEM((EPW,), _i32),
            pltpu.VMEM((EPW,), _f32),
            pltpu.VMEM((4, 128), _i32),
            pltpu.VMEM((CSL,), _f32),
            pltpu.VMEM_SHARED((CPAD,), _f32),
        ],
    )
    return kfn(srcp, dstp, vals, zeros_c)


def _dense_body(p_ref, x_ref, ws1_ref, wn1_ref, b1_ref, ws2_ref, wn2_ref,
                b2_ref, wreg_ref, breg_ref, wf1_ref, bf1_ref, wf2_ref,
                bf2_ref, wfl_ref, bfl_ref, noise_ref,
                c_ref, deg_ref, gen_ref, maskp_ref, dinv_ref):
    C = p_ref[0] + p_ref[1]
    c_ref[...] = C
    x = x_ref[...]
    ones = jnp.ones((N, 1), _f32)
    outdeg = jnp.dot(C, ones, preferred_element_type=_f32)
    indeg = lax.dot_general(C, ones, (((0,), (0,)), ((), ())),
                            preferred_element_type=_f32)
    deg_in = jnp.clip(indeg, 1.0, None)
    neigh1 = lax.dot_general(C, x, (((0,), (0,)), ((), ())),
                             preferred_element_type=_f32) / deg_in
    h1 = jnp.maximum(
        jnp.dot(x, ws1_ref[...], preferred_element_type=_f32)
        + jnp.dot(neigh1, wn1_ref[...], preferred_element_type=_f32)
        + b1_ref[...], 0.0)
    neigh2 = lax.dot_general(C, h1, (((0,), (0,)), ((), ())),
                             preferred_element_type=_f32) / deg_in
    z = (jnp.dot(h1, ws2_ref[...], preferred_element_type=_f32)
         + jnp.dot(neigh2, wn2_ref[...], preferred_element_type=_f32)
         + b2_ref[...])
    degree = jnp.maximum(
        jnp.dot(z, wreg_ref[...], preferred_element_type=_f32)
        + breg_ref[...], 0.0)
    deg_ref[...] = degree
    d = degree
    f = jnp.floor(d)
    frac = d - f
    odd = jnp.floor(f * 0.5) * 2.0 != f
    r = f + jnp.where((frac > 0.5) | ((frac == 0.5) & odd), 1.0, 0.0)
    deg_round = jnp.clip(r, 0.0, float(NP))
    k8 = lax.broadcasted_iota(_i32, (N, 8), 1).astype(_f32)
    maskp_ref[...] = (k8 < deg_round).astype(_f32)
    rowsum = outdeg + deg_round
    dinv_ref[...] = jnp.where(rowsum > 0, lax.rsqrt(rowsum), 0.0)
    g = z + noise_ref[...]
    g = jnp.maximum(jnp.dot(g, wf1_ref[...], preferred_element_type=_f32)
                    + bf1_ref[...], 0.0)
    g = jnp.maximum(jnp.dot(g, wf2_ref[...], preferred_element_type=_f32)
                    + bf2_ref[...], 0.0)
    gen_ref[...] = jnp.tanh(
        jnp.dot(g, wfl_ref[...], preferred_element_type=_f32) + bfl_ref[...])


def _dense(P2, x, Ws1, Wn1, b1, Ws2, Wn2, b2, Wreg, breg,
           Wf1, bf1, Wf2, bf2, Wfl, bfl, noise):
    out_shape = (
        jax.ShapeDtypeStruct((N, N), _f32),
        jax.ShapeDtypeStruct((N, 1), _f32),
        jax.ShapeDtypeStruct((N, NP * IC), _f32),
        jax.ShapeDtypeStruct((N, 8), _f32),
        jax.ShapeDtypeStruct((N, 1), _f32),
    )
    return pl.pallas_call(_dense_body, out_shape=out_shape)(
        P2, x, Ws1, Wn1, b1.reshape(1, -1), Ws2, Wn2, b2.reshape(1, -1),
        Wreg, breg.reshape(1, -1), Wf1, bf1.reshape(1, -1),
        Wf2, bf2.reshape(1, -1), Wfl, bfl.reshape(1, -1), noise)


def _yd_body(x_ref, gen5_ref, wg_ref, dinv_ref, yd_ref):
    wg = wg_ref[...]
    y0 = jnp.dot(x_ref[...], wg, preferred_element_type=_f32)
    y1 = jnp.dot(gen5_ref[...], wg, preferred_element_type=_f32)
    yc = jnp.concatenate([y0, y1], axis=0)
    dcol = jnp.concatenate(
        [dinv_ref[...], jnp.zeros((NT - N, 1), _f32)], axis=0)
    yd_ref[...] = jnp.concatenate(
        [yc, dcol, jnp.zeros((NT, 63), _f32)], axis=1)


def _yd(x, gen5, Wg, dinv):
    return pl.pallas_call(
        _yd_body, out_shape=jax.ShapeDtypeStruct((NT, 128), _f32))(
            x, gen5, Wg, dinv)


WR = 40


def _walk_body(cur_ref, c_ref, m_ref, g_ref, o_ref):
    i = pl.program_id(0)
    rows, mrows, ivals = [], [], []
    for r in range(WR):
        cv = cur_ref[i * WR + r]
        cc = jnp.minimum(cv, N - 1)
        rows.append(c_ref[pl.ds(cc, 1), :])
        mrows.append(m_ref[pl.ds(cc, 1), :])
        ivals.append(cv)
    crows = jnp.concatenate(rows, axis=0)
    mr = jnp.concatenate(mrows, axis=0)
    iv = jnp.concatenate([v.reshape(1, 1) for v in ivals], axis=0)
    valid = iv < N
    lane_lo = lax.broadcasted_iota(_i32, (WR, N), 1)
    onehot = (lane_lo == iv).astype(_f32)
    low_w = jnp.where(valid, crows + onehot, 0.0)
    low = jnp.log(low_w + 1e-12) + g_ref[:, :N]
    ghi = g_ref[:, N:]
    lane_hi = lax.broadcasted_iota(_i32, (WR, NHI), 1)
    dr = jnp.sum(mr, axis=1, keepdims=True).astype(_i32)
    log0 = jnp.log(jnp.zeros((WR, 1), _f32) + 1e-12)
    log1 = jnp.log(jnp.ones((WR, 1), _f32) + 1e-12)
    rel = lane_hi - NP * iv
    in_mend = valid & (rel >= 0) & (rel < dr)
    in_self = jnp.logical_not(valid) & (lane_hi == iv - N)
    val_hi = ghi + jnp.where(in_mend | in_self, log1, log0)
    ml = jnp.max(low, axis=1, keepdims=True)
    mh = jnp.max(val_hi, axis=1, keepdims=True)
    al = jnp.min(jnp.where(low == ml, lane_lo, NT), axis=1, keepdims=True)
    ah = jnp.min(jnp.where(val_hi == mh, lane_hi, NT), axis=1, keepdims=True)
    o_ref[0, :, :] = jnp.where(ml >= mh, al, ah + N)


def _walk_step(cur, C, maskp, G):
    grid_spec = pltpu.PrefetchScalarGridSpec(
        num_scalar_prefetch=1,
        grid=(N // WR,),
        in_specs=[
            pl.BlockSpec((N, N), lambda i, *_: (0, 0)),
            pl.BlockSpec((N, 8), lambda i, *_: (0, 0)),
            pl.BlockSpec((WR, NT), lambda i, *_: (i, 0)),
        ],
        out_specs=pl.BlockSpec((1, WR, 1), lambda i, *_: (i, 0, 0)),
    )
    out = pl.pallas_call(
        _walk_body, grid_spec=grid_spec,
        out_shape=jax.ShapeDtypeStruct((N // WR, WR, 1), _i32),
        compiler_params=pltpu.CompilerParams(
            dimension_semantics=("arbitrary",)))(
            cur, C, maskp, G)
    return out.reshape(N)


def _sc_gather_body(curs_hbm, yd_hbm, cflat_hbm, ydsub_hbm, cvals_hbm,
                    idx3_v, rows_v, fidx_v, cv_v, sem):
    cid = lax.axis_index("c")
    sid = lax.axis_index("s")
    wid = sid * 2 + cid
    bs = wid * NPW
    for v in range(3):
        pltpu.sync_copy(curs_hbm.at[pl.ds(v * NPAD + bs, NPW)], idx3_v.at[v])
    for v in range(3):
        pltpu.async_copy(yd_hbm.at[idx3_v.at[v]], rows_v, sem).wait()
        pltpu.sync_copy(rows_v, ydsub_hbm.at[v, pl.ds(bs, NPW)])
    iota = lax.broadcasted_iota(_i32, (16,), 0)
    for h in range(2):
        s0 = iota + (bs + 16 * h)
        svals = [s0] + [idx3_v[v, pl.ds(16 * h, 16)] for v in range(3)]
        cl = [jnp.minimum(s, N - 1) for s in svals]
        p = 0
        for v in range(3):
            for u in range(4):
                fidx_v[p, pl.ds(16 * h, 16)] = cl[v] * N + cl[u]
                p += 1
    handles = [pltpu.async_copy(cflat_hbm.at[fidx_v.at[p]], cv_v.at[p], sem)
               for p in range(12)]
    for hd in handles:
        hd.wait()
    for p in range(12):
        pltpu.sync_copy(cv_v.at[p], cvals_hbm.at[pl.ds(p * NPAD + bs, NPW)])


def _sc_gather(curs, yd, cflat):
    mesh = plsc.VectorSubcoreMesh(core_axis_name="c", subcore_axis_name="s")
    kfn = pl.kernel(
        _sc_gather_body,
        out_type=(jax.ShapeDtypeStruct((3, NPAD, 128), _f32),
                  jax.ShapeDtypeStruct((12 * NPAD,), _f32)),
        mesh=mesh,
        scratch_types=[
            pltpu.VMEM((3, NPW), _i32),
            pltpu.VMEM((NPW, 128), _f32),
            pltpu.VMEM((12, NPW), _i32),
            pltpu.VMEM((12, NPW), _f32),
            pltpu.SemaphoreType.DMA,
        ],
    )
    return kfn(curs, yd, cflat)


def _final_body(y0_ref, dg0_ref, ys_ref, dgs_ref, cv_ref, s_ref,
                wb_ref, bb_ref, bg_ref, pa_ref, out_ref):
    bg = bg_ref[...]
    pa = pa_ref[0, 0]
    s0 = lax.broadcasted_iota(_i32, (N, 1), 0)
    svals = [s0] + [s_ref[v].astype(_i32) for v in range(3)]
    dgs = [dg0_ref[...]] + [dgs_ref[v] for v in range(3)]
    ys = [y0_ref[...]] + [ys_ref[v] for v in range(3)]
    hsum = jnp.zeros((N, 64), _f32)
    for u in range(4):
        hg = jnp.broadcast_to(bg, (N, 64))
        for v in range(3):
            cval = cv_ref[:, v * 4 + u:v * 4 + u + 1]
            eq = (svals[u] == svals[v]).astype(_f32)
            A = dgs[u] * cval * dgs[v] + eq
            hg = hg + A * ys[v]
        hsum = hsum + jnp.where(hg > 0, hg, pa * hg)
    c = hsum * 0.25
    hmv_pre = ys[3] + bg
    h_mv = jnp.where(hmv_pre > 0, hmv_pre, pa * hmv_pre)
    m = jnp.dot(h_mv, wb_ref[...], preferred_element_type=_f32)
    out_ref[...] = (jnp.sum(m * c, axis=1, keepdims=True) + bb_ref[0, 0])


def _final(y0, dg0, ys3, dgs3, cvals, s3, Wb0, bb, bg, pa):
    return pl.pallas_call(
        _final_body, out_shape=jax.ShapeDtypeStruct((N, 1), _f32))(
            y0, dg0, ys3, dgs3, cvals, s3,
            Wb0, bb.reshape(1, 1), bg.reshape(1, -1), pa.reshape(1, 1))


def kernel(x, edge_index, W_self1, W_neigh1, b1, W_self2, W_neigh2, b2,
           W_reg, b_reg, W_fc1, b_fc1, W_fc2, b_fc2, W_flat, b_flat,
           W_gcn, b_gcn, prelu_a, W_bil, b_bil):
    src = edge_index[0].astype(_i32)
    dst = edge_index[1].astype(_i32)
    pad = EPAD - E
    srcp = jnp.concatenate([src, jnp.zeros((pad,), _i32)])
    dstp = jnp.concatenate([dst, jnp.zeros((pad,), _i32)])
    vals = jnp.concatenate([jnp.ones((E,), _f32), jnp.zeros((pad,), _f32)])
    zeros_c = jnp.zeros((CPAD,), _f32)

    partials = _sc_build_c(srcp, dstp, vals, zeros_c)
    P2 = partials.reshape(2, CPAD)[:, :N * N].reshape(2, N, N)

    noise = jax.random.normal(jax.random.key(7), (N, 64), _f32)
    C, degree, gen_feat, maskp, d_inv = _dense(
        P2, x, W_self1, W_neigh1, b1, W_self2, W_neigh2, b2, W_reg, b_reg,
        W_fc1, b_fc1, W_fc2, b_fc2, W_flat, b_flat, noise)

    gen5 = gen_feat.reshape(NP * N, IC)
    yd = _yd(x, gen5, W_gcn, d_inv)

    cur = jnp.arange(N, dtype=_i32)
    curs = []
    for t in range(1, 4):
        G = jax.random.gumbel(
            jax.random.fold_in(jax.random.key(42), t), (N, NT), _f32)
        cur = _walk_step(cur, C, maskp, G)
        curs.append(cur)

    curs_pad = jnp.concatenate(
        [jnp.concatenate([c, jnp.zeros((NPAD - N,), _i32)]) for c in curs])
    cflat = C.reshape(N * N)
    ydsub, cvals_flat = _sc_gather(curs_pad, yd, cflat)
    cvals = cvals_flat.reshape(12, NPAD)

    y0 = yd[:N, :64]
    dg0 = d_inv
    ys3 = ydsub[:, :N, :64]
    dgs3 = ydsub[:, :N, 64:65]
    s3 = jnp.stack(curs).reshape(3, N, 1)[:, :, :]
    logits = _final(y0, dg0, ys3, dgs3, cvals[:, :N].T, s3,
                    W_bil[0], b_bil, b_gcn, prelu_a)
    return degree, gen_feat, logits


# speedup vs baseline: 4.9171x; 1.0228x over previous
import functools

import jax
import jax.numpy as jnp
from jax import lax
from jax.experimental import pallas as pl
from jax.experimental.pallas import tpu as pltpu
from jax.experimental.pallas import tpu_sc as plsc

N = 1000
E = 16000
IC = 128
NP = 5
NT = N * (1 + NP)
NHI = NT - N

NW = 32
EPAD = 16384
EPW = EPAD // NW
CPAD = 1000448
CSL = CPAD // 16
NPAD = 1024
NPW = NPAD // NW

_f32 = jnp.float32
_i32 = jnp.int32


def _sc_build_c_body(src_hbm, dst_hbm, val_hbm, zeros_hbm, out_hbm,
                     src_v, dst_v, val_v, fidx_v, zbuf_v, csh):
    cid = lax.axis_index("c")
    sid = lax.axis_index("s")
    wid = sid * 2 + cid
    pltpu.sync_copy(zeros_hbm.at[pl.ds(sid * CSL, CSL)], zbuf_v)
    pltpu.sync_copy(zbuf_v, csh.at[pl.ds(sid * CSL, CSL)])
    pltpu.sync_copy(src_hbm.at[pl.ds(wid * EPW, EPW)], src_v)
    pltpu.sync_copy(dst_hbm.at[pl.ds(wid * EPW, EPW)], dst_v)
    pltpu.sync_copy(val_hbm.at[pl.ds(wid * EPW, EPW)], val_v)
    for j in range(EPW // 16):
        sl = pl.ds(j * 16, 16)
        f = src_v[sl] * N + dst_v[sl]
        fidx_v[j // 8, pl.ds((j % 8) * 16, 16)] = f
    plsc.subcore_barrier()
    for ch in range(4):
        pltpu.sync_copy(val_v.at[pl.ds(ch * 128, 128)],
                        csh.at[fidx_v.at[ch]], add=True)
    plsc.subcore_barrier()
    pltpu.sync_copy(csh.at[pl.ds(sid * CSL, CSL)], zbuf_v)
    pltpu.sync_copy(zbuf_v, out_hbm.at[pl.ds(cid * CPAD + sid * CSL, CSL)])


def _sc_build_c(srcp, dstp, vals, zeros_c):
    mesh = plsc.VectorSubcoreMesh(core_axis_name="c", subcore_axis_name="s")
    kfn = pl.kernel(
        _sc_build_c_body,
        out_type=jax.ShapeDtypeStruct((2 * CPAD,), _f32),
        mesh=mesh,
        scratch_types=[
            pltpu.VMEM((EPW,), _i32),
            pltpu.VMEM((EPW,), _i32),
            pltpu.VMEM((EPW,), _f32),
            pltpu.VMEM((4, 128), _i32),
            pltpu.VMEM((CSL,), _f32),
            pltpu.VMEM_SHARED((CPAD,), _f32),
        ],
    )
    return kfn(srcp, dstp, vals, zeros_c)


def _dense_body(p_ref, x_ref, ws1_ref, wn1_ref, b1_ref, ws2_ref, wn2_ref,
                b2_ref, wreg_ref, breg_ref, wf1_ref, bf1_ref, wf2_ref,
                bf2_ref, wfl_ref, bfl_ref, noise_ref,
                c_ref, deg_ref, gen_ref, maskp_ref, dinv_ref):
    C = p_ref[0] + p_ref[1]
    c_ref[...] = C
    x = x_ref[...]
    ones = jnp.ones((N, 1), _f32)
    outdeg = jnp.dot(C, ones, preferred_element_type=_f32)
    indeg = lax.dot_general(C, ones, (((0,), (0,)), ((), ())),
                            preferred_element_type=_f32)
    deg_in = jnp.clip(indeg, 1.0, None)
    neigh1 = lax.dot_general(C, x, (((0,), (0,)), ((), ())),
                             preferred_element_type=_f32) / deg_in
    h1 = jnp.maximum(
        jnp.dot(x, ws1_ref[...], preferred_element_type=_f32)
        + jnp.dot(neigh1, wn1_ref[...], preferred_element_type=_f32)
        + b1_ref[...], 0.0)
    neigh2 = lax.dot_general(C, h1, (((0,), (0,)), ((), ())),
                             preferred_element_type=_f32) / deg_in
    z = (jnp.dot(h1, ws2_ref[...], preferred_element_type=_f32)
         + jnp.dot(neigh2, wn2_ref[...], preferred_element_type=_f32)
         + b2_ref[...])
    degree = jnp.maximum(
        jnp.dot(z, wreg_ref[...], preferred_element_type=_f32)
        + breg_ref[...], 0.0)
    deg_ref[...] = degree
    d = degree
    f = jnp.floor(d)
    frac = d - f
    odd = jnp.floor(f * 0.5) * 2.0 != f
    r = f + jnp.where((frac > 0.5) | ((frac == 0.5) & odd), 1.0, 0.0)
    deg_round = jnp.clip(r, 0.0, float(NP))
    k8 = lax.broadcasted_iota(_i32, (N, 8), 1).astype(_f32)
    maskp_ref[...] = (k8 < deg_round).astype(_f32)
    rowsum = outdeg + deg_round
    dinv_ref[...] = jnp.where(rowsum > 0, lax.rsqrt(rowsum), 0.0)
    g = z + noise_ref[...]
    g = jnp.maximum(jnp.dot(g, wf1_ref[...], preferred_element_type=_f32)
                    + bf1_ref[...], 0.0)
    g = jnp.maximum(jnp.dot(g, wf2_ref[...], preferred_element_type=_f32)
                    + bf2_ref[...], 0.0)
    gen_ref[...] = jnp.tanh(
        jnp.dot(g, wfl_ref[...], preferred_element_type=_f32) + bfl_ref[...])


def _dense(P2, x, Ws1, Wn1, b1, Ws2, Wn2, b2, Wreg, breg,
           Wf1, bf1, Wf2, bf2, Wfl, bfl, noise):
    out_shape = (
        jax.ShapeDtypeStruct((N, N), _f32),
        jax.ShapeDtypeStruct((N, 1), _f32),
        jax.ShapeDtypeStruct((N, NP * IC), _f32),
        jax.ShapeDtypeStruct((N, 8), _f32),
        jax.ShapeDtypeStruct((N, 1), _f32),
    )
    return pl.pallas_call(_dense_body, out_shape=out_shape)(
        P2, x, Ws1, Wn1, b1.reshape(1, -1), Ws2, Wn2, b2.reshape(1, -1),
        Wreg, breg.reshape(1, -1), Wf1, bf1.reshape(1, -1),
        Wf2, bf2.reshape(1, -1), Wfl, bfl.reshape(1, -1), noise)


def _yd_body(x_ref, gen5_ref, wg_ref, dinv_ref, yd_ref):
    wg = wg_ref[...]
    y0 = jnp.dot(x_ref[...], wg, preferred_element_type=_f32)
    y1 = jnp.dot(gen5_ref[...], wg, preferred_element_type=_f32)
    yc = jnp.concatenate([y0, y1], axis=0)
    dcol = jnp.concatenate(
        [dinv_ref[...], jnp.zeros((NT - N, 1), _f32)], axis=0)
    yd_ref[...] = jnp.concatenate(
        [yc, dcol, jnp.zeros((NT, 63), _f32)], axis=1)


def _yd(x, gen5, Wg, dinv):
    return pl.pallas_call(
        _yd_body, out_shape=jax.ShapeDtypeStruct((NT, 128), _f32))(
            x, gen5, Wg, dinv)


WR = 40


def _walk_body(cur_ref, c_ref, m_ref, g_ref, o_ref):
    i = pl.program_id(0)
    rows, mrows, ivals = [], [], []
    for r in range(WR):
        cv = cur_ref[i * WR + r]
        cc = jnp.minimum(cv, N - 1)
        rows.append(c_ref[pl.ds(cc, 1), :])
        mrows.append(m_ref[pl.ds(cc, 1), :])
        ivals.append(cv)
    crows = jnp.concatenate(rows, axis=0)
    mr = jnp.concatenate(mrows, axis=0)
    iv = jnp.concatenate([v.reshape(1, 1) for v in ivals], axis=0)
    valid = iv < N
    lane_lo = lax.broadcasted_iota(_i32, (WR, N), 1)
    onehot = (lane_lo == iv).astype(_f32)
    low_w = jnp.where(valid, crows + onehot, 0.0)
    low = jnp.log(low_w + 1e-12) + g_ref[:, :N]
    ghi = g_ref[:, N:]
    lane_hi = lax.broadcasted_iota(_i32, (WR, NHI), 1)
    dr = jnp.sum(mr, axis=1, keepdims=True).astype(_i32)
    log0 = jnp.log(jnp.zeros((WR, 1), _f32) + 1e-12)
    log1 = jnp.log(jnp.ones((WR, 1), _f32) + 1e-12)
    blo = jnp.where(valid, NP * iv, iv - N)
    bhi = jnp.where(valid, NP * iv + dr, iv - N + 1)
    in_boost = (lane_hi >= blo) & (lane_hi < bhi)
    val_hi = ghi + jnp.where(in_boost, log1, log0)
    ml = jnp.max(low, axis=1, keepdims=True)
    mh = jnp.max(val_hi, axis=1, keepdims=True)
    al = jnp.min(jnp.where(low == ml, lane_lo, NT), axis=1, keepdims=True)
    ah = jnp.min(jnp.where(val_hi == mh, lane_hi, NT), axis=1, keepdims=True)
    o_ref[0, :, :] = jnp.where(ml >= mh, al, ah + N)


def _walk_step(cur, C, maskp, G):
    grid_spec = pltpu.PrefetchScalarGridSpec(
        num_scalar_prefetch=1,
        grid=(N // WR,),
        in_specs=[
            pl.BlockSpec((N, N), lambda i, *_: (0, 0)),
            pl.BlockSpec((N, 8), lambda i, *_: (0, 0)),
            pl.BlockSpec((WR, NT), lambda i, *_: (i, 0)),
        ],
        out_specs=pl.BlockSpec((1, WR, 1), lambda i, *_: (i, 0, 0)),
    )
    out = pl.pallas_call(
        _walk_body, grid_spec=grid_spec,
        out_shape=jax.ShapeDtypeStruct((N // WR, WR, 1), _i32),
        compiler_params=pltpu.CompilerParams(
            dimension_semantics=("arbitrary",)))(
            cur, C, maskp, G)
    return out.reshape(N)


def _sc_gather_body(curs_hbm, yd_hbm, cflat_hbm, ydsub_hbm, cvals_hbm,
                    idx3_v, rows_v, fidx_v, cv_v, sem):
    cid = lax.axis_index("c")
    sid = lax.axis_index("s")
    wid = sid * 2 + cid
    bs = wid * NPW
    for v in range(3):
        pltpu.sync_copy(curs_hbm.at[pl.ds(v * NPAD + bs, NPW)], idx3_v.at[v])
    for v in range(3):
        pltpu.async_copy(yd_hbm.at[idx3_v.at[v]], rows_v, sem).wait()
        pltpu.sync_copy(rows_v, ydsub_hbm.at[v, pl.ds(bs, NPW)])
    iota = lax.broadcasted_iota(_i32, (16,), 0)
    for h in range(2):
        s0 = iota + (bs + 16 * h)
        svals = [s0] + [idx3_v[v, pl.ds(16 * h, 16)] for v in range(3)]
        cl = [jnp.minimum(s, N - 1) for s in svals]
        p = 0
        for v in range(3):
            for u in range(4):
                fidx_v[p, pl.ds(16 * h, 16)] = cl[v] * N + cl[u]
                p += 1
    handles = [pltpu.async_copy(cflat_hbm.at[fidx_v.at[p]], cv_v.at[p], sem)
               for p in range(12)]
    for hd in handles:
        hd.wait()
    for p in range(12):
        pltpu.sync_copy(cv_v.at[p], cvals_hbm.at[pl.ds(p * NPAD + bs, NPW)])


def _sc_gather(curs, yd, cflat):
    mesh = plsc.VectorSubcoreMesh(core_axis_name="c", subcore_axis_name="s")
    kfn = pl.kernel(
        _sc_gather_body,
        out_type=(jax.ShapeDtypeStruct((3, NPAD, 128), _f32),
                  jax.ShapeDtypeStruct((12 * NPAD,), _f32)),
        mesh=mesh,
        scratch_types=[
            pltpu.VMEM((3, NPW), _i32),
            pltpu.VMEM((NPW, 128), _f32),
            pltpu.VMEM((12, NPW), _i32),
            pltpu.VMEM((12, NPW), _f32),
            pltpu.SemaphoreType.DMA,
        ],
    )
    return kfn(curs, yd, cflat)


def _final_body(y0_ref, dg0_ref, ys_ref, dgs_ref, cv_ref, s_ref,
                wb_ref, bb_ref, bg_ref, pa_ref, out_ref):
    bg = bg_ref[...]
    pa = pa_ref[0, 0]
    s0 = lax.broadcasted_iota(_i32, (N, 1), 0)
    svals = [s0] + [s_ref[v].astype(_i32) for v in range(3)]
    dgs = [dg0_ref[...]] + [dgs_ref[v] for v in range(3)]
    ys = [y0_ref[...]] + [ys_ref[v] for v in range(3)]
    hsum = jnp.zeros((N, 64), _f32)
    for u in range(4):
        hg = jnp.broadcast_to(bg, (N, 64))
        for v in range(3):
            cval = cv_ref[:, v * 4 + u:v * 4 + u + 1]
            eq = (svals[u] == svals[v]).astype(_f32)
            A = dgs[u] * cval * dgs[v] + eq
            hg = hg + A * ys[v]
        hsum = hsum + jnp.where(hg > 0, hg, pa * hg)
    c = hsum * 0.25
    hmv_pre = ys[3] + bg
    h_mv = jnp.where(hmv_pre > 0, hmv_pre, pa * hmv_pre)
    m = jnp.dot(h_mv, wb_ref[...], preferred_element_type=_f32)
    out_ref[...] = (jnp.sum(m * c, axis=1, keepdims=True) + bb_ref[0, 0])


def _final(y0, dg0, ys3, dgs3, cvals, s3, Wb0, bb, bg, pa):
    return pl.pallas_call(
        _final_body, out_shape=jax.ShapeDtypeStruct((N, 1), _f32))(
            y0, dg0, ys3, dgs3, cvals, s3,
            Wb0, bb.reshape(1, 1), bg.reshape(1, -1), pa.reshape(1, 1))


def kernel(x, edge_index, W_self1, W_neigh1, b1, W_self2, W_neigh2, b2,
           W_reg, b_reg, W_fc1, b_fc1, W_fc2, b_fc2, W_flat, b_flat,
           W_gcn, b_gcn, prelu_a, W_bil, b_bil):
    src = edge_index[0].astype(_i32)
    dst = edge_index[1].astype(_i32)
    pad = EPAD - E
    srcp = jnp.concatenate([src, jnp.zeros((pad,), _i32)])
    dstp = jnp.concatenate([dst, jnp.zeros((pad,), _i32)])
    vals = jnp.concatenate([jnp.ones((E,), _f32), jnp.zeros((pad,), _f32)])
    zeros_c = jnp.zeros((CPAD,), _f32)

    partials = _sc_build_c(srcp, dstp, vals, zeros_c)
    P2 = partials.reshape(2, CPAD)[:, :N * N].reshape(2, N, N)

    noise = jax.random.normal(jax.random.key(7), (N, 64), _f32)
    C, degree, gen_feat, maskp, d_inv = _dense(
        P2, x, W_self1, W_neigh1, b1, W_self2, W_neigh2, b2, W_reg, b_reg,
        W_fc1, b_fc1, W_fc2, b_fc2, W_flat, b_flat, noise)

    gen5 = gen_feat.reshape(NP * N, IC)
    yd = _yd(x, gen5, W_gcn, d_inv)

    cur = jnp.arange(N, dtype=_i32)
    curs = []
    for t in range(1, 4):
        G = jax.random.gumbel(
            jax.random.fold_in(jax.random.key(42), t), (N, NT), _f32)
        cur = _walk_step(cur, C, maskp, G)
        curs.append(cur)

    curs_pad = jnp.concatenate(
        [jnp.concatenate([c, jnp.zeros((NPAD - N,), _i32)]) for c in curs])
    cflat = C.reshape(N * N)
    ydsub, cvals_flat = _sc_gather(curs_pad, yd, cflat)
    cvals = cvals_flat.reshape(12, NPAD)

    y0 = yd[:N, :64]
    dg0 = d_inv
    ys3 = ydsub[:, :N, :64]
    dgs3 = ydsub[:, :N, 64:65]
    s3 = jnp.stack(curs).reshape(3, N, 1)[:, :, :]
    logits = _final(y0, dg0, ys3, dgs3, cvals[:, :N].T, s3,
                    W_bil[0], b_bil, b_gcn, prelu_a)
    return degree, gen_feat, logits
